# baseline (speedup 1.0000x reference)
from itertools import combinations

import jax
import jax.numpy as jnp
from jax.experimental import pallas as pl

B, F, D = 16384, 26, 16
P = F * (F - 1) // 2
BC = 1024

_ROW_OFF = [0]
for _f in range(F - 1):
    _ROW_OFF.append(_ROW_OFF[-1] + (F - 1 - _f))


def _stats_body(xt_ref, prod_ref, s_ref):
    i = pl.program_id(0)
    x3 = xt_ref[...].T.reshape(F, D, BC)
    s1_parts, s2_parts = [], []
    for f in range(F - 1):
        r = F - 1 - f
        part = jnp.sum(x3[f:f + 1] * x3[f + 1:], axis=1)
        prod_ref[_ROW_OFF[f]:_ROW_OFF[f] + r, :] = part
        s1_parts.append(jnp.sum(part, axis=1, keepdims=True))
        s2_parts.append(jnp.sum(part * part, axis=1, keepdims=True))
    s1 = jnp.concatenate(s1_parts, axis=0)
    s2 = jnp.concatenate(s2_parts, axis=0)
    s = jnp.concatenate([s1, s2], axis=1)

    @pl.when(i == 0)
    def _():
        s_ref[...] = jnp.zeros_like(s_ref)

    s_ref[...] += s


def _out_body(s_ref, alpha_ref, prod_ref, out_ref):
    s = s_ref[...]
    m = s[:, 0:1] * (1.0 / B)
    var = s[:, 1:2] * (1.0 / B) - m * m
    w = jnp.tanh(alpha_ref[...]) * jax.lax.rsqrt(var + 1e-3)
    c = jnp.sum(w * m)
    out_ref[...] = jnp.sum(prod_ref[...] * w, axis=0, keepdims=True) - c


def kernel(embed_matrix, alpha, feat_i, feat_j):
    del feat_i, feat_j
    xt = embed_matrix.reshape(B, F * D)
    nb = B // BC
    prod, s = pl.pallas_call(
        _stats_body,
        grid=(nb,),
        in_specs=[pl.BlockSpec((BC, F * D), lambda i: (i, 0))],
        out_specs=[
            pl.BlockSpec((P, BC), lambda i: (0, i)),
            pl.BlockSpec((P, 2), lambda i: (0, 0)),
        ],
        out_shape=[
            jax.ShapeDtypeStruct((P, B), jnp.float32),
            jax.ShapeDtypeStruct((P, 2), jnp.float32),
        ],
    )(xt)
    out = pl.pallas_call(
        _out_body,
        grid=(nb,),
        in_specs=[
            pl.BlockSpec((P, 2), lambda i: (0, 0)),
            pl.BlockSpec((P, 1), lambda i: (0, 0)),
            pl.BlockSpec((P, BC), lambda i: (0, i)),
        ],
        out_specs=pl.BlockSpec((1, BC), lambda i: (0, i)),
        out_shape=jax.ShapeDtypeStruct((1, B), jnp.float32),
    )(s, alpha.reshape(P, 1), prod)
    return out.reshape(B, 1)


# speedup vs baseline: 1.7199x; 1.7199x over previous
from itertools import combinations

import jax
import jax.numpy as jnp
from jax.experimental import pallas as pl

B, F, D = 16384, 26, 16
P = F * (F - 1) // 2
BC = 1024

_ROW_OFF = [0]
for _f in range(F - 1):
    _ROW_OFF.append(_ROW_OFF[-1] + (F - 1 - _f))


_PAIRS = list(combinations(range(F), 2))


def _stats_body(xt_ref, prod_ref, s_ref):
    i = pl.program_id(0)
    s1_parts, s2_parts = [], []
    hs, row0 = [], 0
    for fi, gj in _PAIRS:
        q = xt_ref[fi * D:(fi + 1) * D, :] * xt_ref[gj * D:(gj + 1) * D, :]
        hs.append(q[0:8] + q[8:16])
        if len(hs) == 8 or row0 + len(hs) == P:
            n = len(hs)
            H = jnp.stack(hs, axis=0)
            S = jnp.sum(jnp.swapaxes(H, 0, 1), axis=0)
            prod_ref[row0:row0 + n, :] = S
            s1_parts.append(jnp.sum(S, axis=1, keepdims=True))
            s2_parts.append(jnp.sum(S * S, axis=1, keepdims=True))
            hs = []
            row0 += n
    s1 = jnp.concatenate(s1_parts, axis=0)
    s2 = jnp.concatenate(s2_parts, axis=0)
    s = jnp.concatenate([s1, s2], axis=1)

    @pl.when(i == 0)
    def _():
        s_ref[...] = jnp.zeros_like(s_ref)

    s_ref[...] += s


def _out_body(s_ref, alpha_ref, prod_ref, out_ref):
    s = s_ref[...]
    m = s[:, 0:1] * (1.0 / B)
    var = s[:, 1:2] * (1.0 / B) - m * m
    w = jnp.tanh(alpha_ref[...]) * jax.lax.rsqrt(var + 1e-3)
    c = jnp.sum(w * m)
    out_ref[...] = jnp.sum(prod_ref[...] * w, axis=0, keepdims=True) - c


def kernel(embed_matrix, alpha, feat_i, feat_j):
    del feat_i, feat_j
    xt = embed_matrix.reshape(B, F * D).T
    nb = B // BC
    prod, s = pl.pallas_call(
        _stats_body,
        grid=(nb,),
        in_specs=[pl.BlockSpec((F * D, BC), lambda i: (0, i))],
        out_specs=[
            pl.BlockSpec((P, BC), lambda i: (0, i)),
            pl.BlockSpec((P, 2), lambda i: (0, 0)),
        ],
        out_shape=[
            jax.ShapeDtypeStruct((P, B), jnp.float32),
            jax.ShapeDtypeStruct((P, 2), jnp.float32),
        ],
    )(xt)
    out = pl.pallas_call(
        _out_body,
        grid=(nb,),
        in_specs=[
            pl.BlockSpec((P, 2), lambda i: (0, 0)),
            pl.BlockSpec((P, 1), lambda i: (0, 0)),
            pl.BlockSpec((P, BC), lambda i: (0, i)),
        ],
        out_specs=pl.BlockSpec((1, BC), lambda i: (0, i)),
        out_shape=jax.ShapeDtypeStruct((1, B), jnp.float32),
    )(s, alpha.reshape(P, 1), prod)
    return out.reshape(B, 1)
